# baseline (speedup 1.0000x reference)
import functools

import jax
import jax.numpy as jnp
from jax import lax
from jax.experimental import pallas as pl
from jax.experimental.pallas import tpu as pltpu
from jax.experimental.pallas import tpu_sc as plsc

T_K = 10.0
_ROW_BLK = 128
_PAD_VAL = 1.0e4
_LANES = 16
_GQ = 8
_Q_SC = 14016
_Q_CHUNK = 2048


def _stats_body(feat_ref, mus_sc_ref, sigs_sc_ref,
                am_ref, as_ref, amsp_ref, assp_ref, sgp2_ref, cq2_ref, *, d):
    i = pl.program_id(0)

    @pl.when(i == 0)
    def _prep_queue():
        mu = mus_sc_ref[...]
        sgp = sigs_sc_ref[...] - 1.0
        sgp2_ref[...] = sgp
        cq2_ref[...] = mu * mu + sgp * sgp

    f = feat_ref[...]
    m = jnp.mean(f, axis=1, keepdims=True)
    c = f - m
    var = jnp.sum(c * c, axis=1, keepdims=True) / (d - 1)
    sp = jnp.sqrt(var) - 1.0
    am = -2.0 * m
    asv = -2.0 * sp
    am_ref[...] = am[:, 0]
    as_ref[...] = asv[:, 0]
    rows = f.shape[0]
    amsp_ref[...] = jnp.broadcast_to(am, (rows, _LANES))
    assp_ref[...] = jnp.broadcast_to(asv, (rows, _LANES))


def _tc_knn_body(am_ref, as_ref, mus_ref, sig_ref, out_ref, *, q_pad):
    rows = am_ref.shape[0]
    mnegb = jnp.broadcast_to(am_ref[...][:, None], (rows, 128))
    snegb = jnp.broadcast_to(as_ref[...][:, None], (rows, 128))
    n_steps = q_pad // 128

    def qstep(i, best):
        mu = mus_ref[0, pl.ds(i * 128, 128)][None, :]
        sgp = sig_ref[0, pl.ds(i * 128, 128)][None, :] - 1.0
        cq = mu * mu + sgp * sgp
        t = mnegb * mu + (snegb * sgp + cq)
        return jnp.minimum(best, t)

    best0 = jnp.full((rows, 128), jnp.inf, dtype=jnp.float32)
    best = lax.fori_loop(0, n_steps, qstep, best0)
    out_ref[...] = jnp.min(best, axis=1)


def _merge_body(am_ref, as_ref, sc_ref, tc_ref, out_ref):
    am = am_ref[...]
    asv = as_ref[...]
    best_sc = jnp.min(jnp.min(sc_ref[...], axis=0), axis=1)
    best = jnp.minimum(best_sc, tc_ref[...])
    dist2 = jnp.maximum(best + 0.25 * (am * am + asv * asv), 0.0)
    out_ref[...] = jnp.exp(-T_K * jnp.sqrt(dist2))


def _make_sc_knn(b, qh, n_sub):
    qb = b // n_sub
    nch = qh // _LANES
    mesh = plsc.VectorSubcoreMesh(core_axis_name="c", subcore_axis_name="s")

    @functools.partial(
        pl.kernel, mesh=mesh,
        out_type=jax.ShapeDtypeStruct((2, b, _LANES), jnp.float32),
        scratch_types=[
            pltpu.VMEM((qh,), jnp.float32),
            pltpu.VMEM((qh,), jnp.float32),
            pltpu.VMEM((qh,), jnp.float32),
            pltpu.VMEM((qb, _LANES), jnp.float32),
            pltpu.VMEM((qb, _LANES), jnp.float32),
            pltpu.VMEM((qb, _LANES), jnp.float32),
            pltpu.SemaphoreType.DMA,
            pltpu.SemaphoreType.DMA,
            pltpu.SemaphoreType.DMA,
            pltpu.SemaphoreType.DMA,
            pltpu.SemaphoreType.DMA,
        ],
    )
    def sc_knn(mus_hbm, sgp_hbm, cq_hbm, amsp_hbm, assp_hbm, out_hbm,
               mu_v, sg_v, cq_v, am_v, as_v, red_v, s0, s1, s2, s3, s4):
        cid = lax.axis_index("c")
        sid = lax.axis_index("s")
        base = sid * qb
        c0 = pltpu.async_copy(mus_hbm.at[cid], mu_v, s0)
        c1 = pltpu.async_copy(sgp_hbm.at[cid], sg_v, s1)
        c2 = pltpu.async_copy(cq_hbm.at[cid], cq_v, s2)
        c3 = pltpu.async_copy(amsp_hbm.at[pl.ds(base, qb), :], am_v, s3)
        c4 = pltpu.async_copy(assp_hbm.at[pl.ds(base, qb), :], as_v, s4)
        c0.wait()
        c1.wait()
        c2.wait()
        c3.wait()
        c4.wait()

        for g in range(qb // _GQ):
            ams = [am_v[g * _GQ + j, :] for j in range(_GQ)]
            ass = [as_v[g * _GQ + j, :] for j in range(_GQ)]

            def body(k, accs):
                mu16 = mu_v[pl.ds(k * _LANES, _LANES)]
                sg16 = sg_v[pl.ds(k * _LANES, _LANES)]
                cq16 = cq_v[pl.ds(k * _LANES, _LANES)]
                return tuple(
                    jnp.minimum(accs[j], cq16 + ams[j] * mu16 + ass[j] * sg16)
                    for j in range(_GQ))

            acc0 = tuple(jnp.full((_LANES,), jnp.inf, jnp.float32)
                         for _ in range(_GQ))
            accs = lax.fori_loop(0, nch, body, acc0, unroll=2)
            for j in range(_GQ):
                red_v[g * _GQ + j, :] = accs[j]

        pltpu.sync_copy(red_v, out_hbm.at[cid, pl.ds(base, qb), :])

    return sc_knn


def kernel(features, labels, pred, confidence, queue_mus, queue_sigmas):
    del labels, pred, confidence
    b, d = features.shape
    q = queue_mus.shape[0]
    n_sub = 16
    qh = _Q_SC // 2
    q_tc = q - _Q_SC
    q_tc_pad = ((q_tc + _Q_CHUNK - 1) // _Q_CHUNK) * _Q_CHUNK

    mus_sc = queue_mus[:_Q_SC].reshape(2, qh)
    sigs_sc = queue_sigmas[:_Q_SC].reshape(2, qh)
    mus_tc = jnp.pad(queue_mus[_Q_SC:], (0, q_tc_pad - q_tc),
                     constant_values=_PAD_VAL)[None, :]
    sigs_tc = jnp.pad(queue_sigmas[_Q_SC:], (0, q_tc_pad - q_tc),
                      constant_values=_PAD_VAL)[None, :]

    am, asv, amsp, assp, sgp_sc, cq_sc = pl.pallas_call(
        functools.partial(_stats_body, d=d),
        grid=(b // _ROW_BLK,),
        in_specs=[pl.BlockSpec((_ROW_BLK, d), lambda i: (i, 0)),
                  pl.BlockSpec((2, qh), lambda i: (0, 0)),
                  pl.BlockSpec((2, qh), lambda i: (0, 0))],
        out_specs=[pl.BlockSpec((_ROW_BLK,), lambda i: (i,)),
                   pl.BlockSpec((_ROW_BLK,), lambda i: (i,)),
                   pl.BlockSpec((_ROW_BLK, _LANES), lambda i: (i, 0)),
                   pl.BlockSpec((_ROW_BLK, _LANES), lambda i: (i, 0)),
                   pl.BlockSpec((2, qh), lambda i: (0, 0)),
                   pl.BlockSpec((2, qh), lambda i: (0, 0))],
        out_shape=[jax.ShapeDtypeStruct((b,), jnp.float32),
                   jax.ShapeDtypeStruct((b,), jnp.float32),
                   jax.ShapeDtypeStruct((b, _LANES), jnp.float32),
                   jax.ShapeDtypeStruct((b, _LANES), jnp.float32),
                   jax.ShapeDtypeStruct((2, qh), jnp.float32),
                   jax.ShapeDtypeStruct((2, qh), jnp.float32)],
    )(features, mus_sc, sigs_sc)

    part_sc = _make_sc_knn(b, qh, n_sub)(mus_sc, sgp_sc, cq_sc, amsp, assp)

    part_tc = pl.pallas_call(
        functools.partial(_tc_knn_body, q_pad=q_tc_pad),
        grid=(b // _ROW_BLK,),
        in_specs=[pl.BlockSpec((_ROW_BLK,), lambda i: (i,)),
                  pl.BlockSpec((_ROW_BLK,), lambda i: (i,)),
                  pl.BlockSpec((1, q_tc_pad), lambda i: (0, 0)),
                  pl.BlockSpec((1, q_tc_pad), lambda i: (0, 0))],
        out_specs=pl.BlockSpec((_ROW_BLK,), lambda i: (i,)),
        out_shape=jax.ShapeDtypeStruct((b,), jnp.float32),
    )(am, asv, mus_tc, sigs_tc)

    out = pl.pallas_call(
        _merge_body,
        in_specs=[pl.BlockSpec((b,), lambda: (0,)),
                  pl.BlockSpec((b,), lambda: (0,)),
                  pl.BlockSpec((2, b, _LANES), lambda: (0, 0, 0)),
                  pl.BlockSpec((b,), lambda: (0,))],
        out_specs=pl.BlockSpec((b,), lambda: (0,)),
        out_shape=jax.ShapeDtypeStruct((b,), jnp.float32),
    )(am, asv, part_sc, part_tc)
    return out


# speedup vs baseline: 1.0667x; 1.0667x over previous
import functools

import jax
import jax.numpy as jnp
from jax import lax
from jax.experimental import pallas as pl
from jax.experimental.pallas import tpu as pltpu
from jax.experimental.pallas import tpu_sc as plsc

T_K = 10.0
_ROW_BLK = 128
_PAD_VAL = 1.0e4
_LANES = 16
_GQ = 8
_Q_SC = 16000
_Q_CHUNK = 2048


def _stats_body(feat_ref, am_ref, as_ref, amsp_ref, assp_ref, *, d):
    f = feat_ref[...]
    m = jnp.mean(f, axis=1, keepdims=True)
    c = f - m
    var = jnp.sum(c * c, axis=1, keepdims=True) / (d - 1)
    sp = jnp.sqrt(var) - 1.0
    am = -2.0 * m
    asv = -2.0 * sp
    am_ref[...] = am[:, 0]
    as_ref[...] = asv[:, 0]
    rows = f.shape[0]
    amsp_ref[...] = jnp.broadcast_to(am, (rows, _LANES))
    assp_ref[...] = jnp.broadcast_to(asv, (rows, _LANES))


def _tc_knn_body(am_ref, as_ref, mus_ref, sig_ref, out_ref, *, q_pad):
    rows = am_ref.shape[0]
    mnegb = jnp.broadcast_to(am_ref[...][:, None], (rows, 128))
    snegb = jnp.broadcast_to(as_ref[...][:, None], (rows, 128))
    n_steps = q_pad // 128

    def qstep(i, best):
        mu = mus_ref[0, pl.ds(i * 128, 128)][None, :]
        sgp = sig_ref[0, pl.ds(i * 128, 128)][None, :] - 1.0
        cq = mu * mu + sgp * sgp
        t = mnegb * mu + (snegb * sgp + cq)
        return jnp.minimum(best, t)

    best0 = jnp.full((rows, 128), jnp.inf, dtype=jnp.float32)
    best = lax.fori_loop(0, n_steps, qstep, best0)
    out_ref[...] = jnp.min(best, axis=1)


def _merge_body(am_ref, as_ref, sc_ref, tc_ref, out_ref):
    am = am_ref[...]
    asv = as_ref[...]
    best_sc = jnp.min(jnp.min(sc_ref[...], axis=0), axis=1)
    best = jnp.minimum(best_sc, tc_ref[...])
    dist2 = jnp.maximum(best + 0.25 * (am * am + asv * asv), 0.0)
    out_ref[...] = jnp.exp(-T_K * jnp.sqrt(dist2))


def _make_sc_knn(b, qh, n_sub):
    qb = b // n_sub
    nch = qh // _LANES
    mesh = plsc.VectorSubcoreMesh(core_axis_name="c", subcore_axis_name="s")

    @functools.partial(
        pl.kernel, mesh=mesh,
        out_type=jax.ShapeDtypeStruct((2, b, _LANES), jnp.float32),
        scratch_types=[
            pltpu.VMEM((qh,), jnp.float32),
            pltpu.VMEM((qh,), jnp.float32),
            pltpu.VMEM((qh,), jnp.float32),
            pltpu.VMEM((qb, _LANES), jnp.float32),
            pltpu.VMEM((qb, _LANES), jnp.float32),
            pltpu.VMEM((qb, _LANES), jnp.float32),
        ],
    )
    def sc_knn(mus_hbm, sigs_hbm, amsp_hbm, assp_hbm, out_hbm,
               mu_v, sg_v, cq_v, am_v, as_v, red_v):
        cid = lax.axis_index("c")
        sid = lax.axis_index("s")
        base = sid * qb
        pltpu.sync_copy(mus_hbm.at[cid], mu_v)
        pltpu.sync_copy(sigs_hbm.at[cid], sg_v)
        pltpu.sync_copy(amsp_hbm.at[pl.ds(base, qb), :], am_v)
        pltpu.sync_copy(assp_hbm.at[pl.ds(base, qb), :], as_v)

        def prep(k, carry):
            mu16 = mu_v[pl.ds(k * _LANES, _LANES)]
            sg16 = sg_v[pl.ds(k * _LANES, _LANES)] - 1.0
            sg_v[pl.ds(k * _LANES, _LANES)] = sg16
            cq_v[pl.ds(k * _LANES, _LANES)] = mu16 * mu16 + sg16 * sg16
            return carry

        lax.fori_loop(0, nch, prep, 0)

        for g in range(qb // _GQ):
            ams = [am_v[g * _GQ + j, :] for j in range(_GQ)]
            ass = [as_v[g * _GQ + j, :] for j in range(_GQ)]

            def body(k, accs):
                mu16 = mu_v[pl.ds(k * _LANES, _LANES)]
                sg16 = sg_v[pl.ds(k * _LANES, _LANES)]
                cq16 = cq_v[pl.ds(k * _LANES, _LANES)]
                return tuple(
                    jnp.minimum(accs[j], cq16 + ams[j] * mu16 + ass[j] * sg16)
                    for j in range(_GQ))

            acc0 = tuple(jnp.full((_LANES,), jnp.inf, jnp.float32)
                         for _ in range(_GQ))
            accs = lax.fori_loop(0, nch, body, acc0, unroll=2)
            for j in range(_GQ):
                red_v[g * _GQ + j, :] = accs[j]

        pltpu.sync_copy(red_v, out_hbm.at[cid, pl.ds(base, qb), :])

    return sc_knn


def kernel(features, labels, pred, confidence, queue_mus, queue_sigmas):
    del labels, pred, confidence
    b, d = features.shape
    q = queue_mus.shape[0]
    n_sub = 16
    qh = _Q_SC // 2
    q_tc = q - _Q_SC
    q_tc_pad = ((q_tc + _Q_CHUNK - 1) // _Q_CHUNK) * _Q_CHUNK

    mus_sc = queue_mus[:_Q_SC].reshape(2, qh)
    sigs_sc = queue_sigmas[:_Q_SC].reshape(2, qh)
    mus_tc = jnp.pad(queue_mus[_Q_SC:], (0, q_tc_pad - q_tc),
                     constant_values=_PAD_VAL)[None, :]
    sigs_tc = jnp.pad(queue_sigmas[_Q_SC:], (0, q_tc_pad - q_tc),
                      constant_values=_PAD_VAL)[None, :]

    am, asv, amsp, assp = pl.pallas_call(
        functools.partial(_stats_body, d=d),
        grid=(b // _ROW_BLK,),
        in_specs=[pl.BlockSpec((_ROW_BLK, d), lambda i: (i, 0))],
        out_specs=[pl.BlockSpec((_ROW_BLK,), lambda i: (i,)),
                   pl.BlockSpec((_ROW_BLK,), lambda i: (i,)),
                   pl.BlockSpec((_ROW_BLK, _LANES), lambda i: (i, 0)),
                   pl.BlockSpec((_ROW_BLK, _LANES), lambda i: (i, 0))],
        out_shape=[jax.ShapeDtypeStruct((b,), jnp.float32),
                   jax.ShapeDtypeStruct((b,), jnp.float32),
                   jax.ShapeDtypeStruct((b, _LANES), jnp.float32),
                   jax.ShapeDtypeStruct((b, _LANES), jnp.float32)],
    )(features)

    part_sc = _make_sc_knn(b, qh, n_sub)(mus_sc, sigs_sc, amsp, assp)

    part_tc = pl.pallas_call(
        functools.partial(_tc_knn_body, q_pad=q_tc_pad),
        grid=(b // _ROW_BLK,),
        in_specs=[pl.BlockSpec((_ROW_BLK,), lambda i: (i,)),
                  pl.BlockSpec((_ROW_BLK,), lambda i: (i,)),
                  pl.BlockSpec((1, q_tc_pad), lambda i: (0, 0)),
                  pl.BlockSpec((1, q_tc_pad), lambda i: (0, 0))],
        out_specs=pl.BlockSpec((_ROW_BLK,), lambda i: (i,)),
        out_shape=jax.ShapeDtypeStruct((b,), jnp.float32),
    )(am, asv, mus_tc, sigs_tc)

    out = pl.pallas_call(
        _merge_body,
        in_specs=[pl.BlockSpec((b,), lambda: (0,)),
                  pl.BlockSpec((b,), lambda: (0,)),
                  pl.BlockSpec((2, b, _LANES), lambda: (0, 0, 0)),
                  pl.BlockSpec((b,), lambda: (0,))],
        out_specs=pl.BlockSpec((b,), lambda: (0,)),
        out_shape=jax.ShapeDtypeStruct((b,), jnp.float32),
    )(am, asv, part_sc, part_tc)
    return out
